# baseline (speedup 1.0000x reference)
import functools

import jax
import jax.numpy as jnp
from jax import lax
from jax.experimental import pallas as pl
from jax.experimental.pallas import tpu as pltpu
from jax.experimental.pallas import tpu_sc as plsc

N = 10000
NP = 10240
F_IN = 128
C = 64
G = 64
NC = 2
NS = 16
NW = NC * NS
RPT = NP // NS
CHUNK = 128
NCH = 80
E_PAD = NW * NCH * CHUNK

_mesh = plsc.VectorSubcoreMesh(core_axis_name="c", subcore_axis_name="s")
_sc_params = pltpu.CompilerParams(use_tc_tiling_on_sc=False)


@functools.partial(
    pl.kernel,
    mesh=_mesh,
    out_type=jax.ShapeDtypeStruct((NC * NP, 16), jnp.float32),
    compiler_params=_sc_params,
    scratch_types=[
        pltpu.VMEM((NCH, CHUNK), jnp.int32),
        pltpu.VMEM((CHUNK, 16), jnp.float32),
        pltpu.VMEM_SHARED((NP, 16), jnp.float32),
    ],
)
def _deg_kernel(dst_hbm, ones_hbm, zeros_hbm, out_hbm, dst_v, ones_v, hist):
    c = lax.axis_index("c")
    s = lax.axis_index("s")
    wid = s * NC + c
    pltpu.sync_copy(dst_hbm.at[wid], dst_v)
    pltpu.sync_copy(ones_hbm, ones_v)
    pltpu.sync_copy(zeros_hbm, hist.at[pl.ds(s * RPT, RPT)])
    plsc.subcore_barrier()

    def body(j, carry):
        pltpu.sync_copy(ones_v, hist.at[dst_v.at[j]], add=True)
        return carry

    lax.fori_loop(0, NCH, body, 0)
    plsc.subcore_barrier()
    pltpu.sync_copy(hist.at[pl.ds(s * RPT, RPT)],
                    out_hbm.at[pl.ds(c * NP + s * RPT, RPT)])


GRP = 4
NGRP = NCH // GRP
NCHI = NCH + GRP


@functools.partial(
    pl.kernel,
    mesh=_mesh,
    out_type=jax.ShapeDtypeStruct((NC * NP, C), jnp.float32),
    compiler_params=_sc_params,
    scratch_types=[
        pltpu.VMEM((NCHI, CHUNK), jnp.int32),
        pltpu.VMEM((NCH, CHUNK), jnp.int32),
        pltpu.VMEM((CHUNK, C), jnp.float32),
        pltpu.VMEM((CHUNK, C), jnp.float32),
        pltpu.VMEM_SHARED((NP, C), jnp.float32),
        pltpu.VMEM_SHARED((NP, C), jnp.float32),
        pltpu.SemaphoreType.DMA,
        pltpu.SemaphoreType.DMA,
        pltpu.SemaphoreType.DMA,
        pltpu.SemaphoreType.DMA,
    ],
)
def _scatter_kernel(hs_hbm, src_hbm, dst_hbm, zeros_hbm, out_hbm,
                    src_v, dst_v, buf_a, buf_b, hs_s, acc,
                    sga, sgb, ssa, ssb):
    c = lax.axis_index("c")
    s = lax.axis_index("s")
    wid = s * NC + c
    pltpu.sync_copy(src_hbm.at[wid], src_v)
    pltpu.sync_copy(dst_hbm.at[wid], dst_v)
    pltpu.sync_copy(hs_hbm.at[pl.ds(s * RPT, RPT)],
                    hs_s.at[pl.ds(s * RPT, RPT)])
    pltpu.sync_copy(zeros_hbm, acc.at[pl.ds(s * RPT, RPT)])
    plsc.subcore_barrier()

    def start_g(buf, sem, j):
        pltpu.make_async_copy(hs_s.at[src_v.at[j]], buf, sem).start()

    def wait_g(buf, sem):
        pltpu.make_async_copy(hs_s.at[src_v.at[0]], buf, sem).wait()

    def start_s(buf, sem, j):
        pltpu.make_async_copy(buf, acc.at[dst_v.at[j]], sem).start(add=True)

    def wait_s(buf, sem):
        pltpu.make_async_copy(buf, acc.at[dst_v.at[0]], sem).wait()

    start_g(buf_a, sga, 0)
    start_g(buf_b, sgb, 1)

    def body(i, carry):
        j = 2 * i
        wait_g(buf_a, sga)
        start_s(buf_a, ssa, j)
        wait_g(buf_b, sgb)
        start_s(buf_b, ssb, j + 1)
        wait_s(buf_a, ssa)
        start_g(buf_a, sga, j + 2)
        wait_s(buf_b, ssb)
        start_g(buf_b, sgb, j + 3)
        return carry

    lax.fori_loop(0, NCH // 2, body, 0)
    wait_g(buf_a, sga)
    wait_g(buf_b, sgb)
    plsc.subcore_barrier()
    pltpu.sync_copy(acc.at[pl.ds(s * RPT, RPT)],
                    out_hbm.at[pl.ds(c * NP + s * RPT, RPT)])


def _dis(hist_ref):
    deg = hist_ref[0, :, 0:1] + hist_ref[1, :, 0:1] + 1.0
    return lax.rsqrt(deg)


def _layer1_body(x_ref, hist_ref, w_ref, o_ref):
    h = jnp.dot(x_ref[...], w_ref[...], preferred_element_type=jnp.float32)
    o_ref[...] = h * _dis(hist_ref)


def _mid_body(t_ref, hsp_ref, hist_ref, b_ref, w_ref, o_ref):
    dis = _dis(hist_ref)
    agg = dis * (t_ref[0] + t_ref[1] + hsp_ref[...]) + b_ref[...]
    h = jnp.maximum(agg, 0.0)
    o_ref[...] = jnp.dot(h, w_ref[...], preferred_element_type=jnp.float32) * dis


def _final_body(t_ref, hs3_ref, hist_ref, b3_ref, seg_ref,
                wl1_ref, bl1_ref, wl2_ref, bl2_ref, o_ref, p_acc):
    i = pl.program_id(0)

    @pl.when(i == 0)
    def _():
        p_acc[...] = jnp.zeros_like(p_acc)

    dis = _dis(hist_ref)
    h3 = dis * (t_ref[0] + t_ref[1] + hs3_ref[...]) + b3_ref[...]
    onehot = (seg_ref[...] == lax.broadcasted_iota(jnp.int32, (RPT, G), 1)
              ).astype(jnp.float32)
    p_acc[...] += lax.dot_general(onehot, h3, (((0,), (0,)), ((), ())),
                                  preferred_element_type=jnp.float32)

    @pl.when(i == pl.num_programs(0) - 1)
    def _():
        p = p_acc[...]
        pr = jnp.maximum(
            jnp.dot(p, wl1_ref[...], preferred_element_type=jnp.float32)
            + bl1_ref[...], 0.0)
        o_ref[...] = (jnp.dot(pr, wl2_ref[...],
                              preferred_element_type=jnp.float32)
                      + bl2_ref[...])


def _layer1(xp, hist, W1):
    return pl.pallas_call(
        _layer1_body,
        grid=(NS,),
        in_specs=[
            pl.BlockSpec((RPT, F_IN), lambda i: (i, 0)),
            pl.BlockSpec((2, RPT, 16), lambda i: (0, i, 0)),
            pl.BlockSpec((F_IN, C), lambda i: (0, 0)),
        ],
        out_specs=pl.BlockSpec((RPT, C), lambda i: (i, 0)),
        out_shape=jax.ShapeDtypeStruct((NP, C), jnp.float32),
    )(xp, hist, W1)


def _mid(t, hsp, hist, bias, W):
    return pl.pallas_call(
        _mid_body,
        grid=(NS,),
        in_specs=[
            pl.BlockSpec((2, RPT, C), lambda i: (0, i, 0)),
            pl.BlockSpec((RPT, C), lambda i: (i, 0)),
            pl.BlockSpec((2, RPT, 16), lambda i: (0, i, 0)),
            pl.BlockSpec((1, C), lambda i: (0, 0)),
            pl.BlockSpec((C, C), lambda i: (0, 0)),
        ],
        out_specs=pl.BlockSpec((RPT, C), lambda i: (i, 0)),
        out_shape=jax.ShapeDtypeStruct((NP, C), jnp.float32),
    )(t, hsp, hist, bias, W)


def _final(t, hs3, hist, b3, segp, Wl1, bl1, Wl2, bl2):
    return pl.pallas_call(
        _final_body,
        grid=(NS,),
        in_specs=[
            pl.BlockSpec((2, RPT, C), lambda i: (0, i, 0)),
            pl.BlockSpec((RPT, C), lambda i: (i, 0)),
            pl.BlockSpec((2, RPT, 16), lambda i: (0, i, 0)),
            pl.BlockSpec((1, C), lambda i: (0, 0)),
            pl.BlockSpec((RPT, 1), lambda i: (i, 0)),
            pl.BlockSpec((C, 32), lambda i: (0, 0)),
            pl.BlockSpec((1, 32), lambda i: (0, 0)),
            pl.BlockSpec((32, 1), lambda i: (0, 0)),
            pl.BlockSpec((1, 1), lambda i: (0, 0)),
        ],
        out_specs=pl.BlockSpec((G, 1), lambda i: (0, 0)),
        out_shape=jax.ShapeDtypeStruct((G, 1), jnp.float32),
        scratch_shapes=[pltpu.VMEM((G, C), jnp.float32)],
    )(t, hs3, hist, b3, segp, Wl1, bl1, Wl2, bl2)


def kernel(x, e, b, W1, b1, W2, b2, W3, b3, Wl1, bl1, Wl2, bl2):
    E = e.shape[1]
    xp = jnp.pad(x, ((0, NP - N), (0, 0)))
    pad = jnp.full((E_PAD - E,), N, jnp.int32)
    srcp = jnp.concatenate([e[0], pad]).reshape(NW, NCH, CHUNK)
    srcp = jnp.concatenate(
        [srcp, jnp.full((NW, NCHI - NCH, CHUNK), N, jnp.int32)], axis=1)
    dstp = jnp.concatenate([e[1], pad]).reshape(NW, NCH, CHUNK)
    segp = jnp.concatenate([b, jnp.full((NP - N,), G, jnp.int32)]
                           ).reshape(NP, 1)
    ones16 = jnp.ones((CHUNK, 16), jnp.float32)
    zer16 = jnp.zeros((RPT, 16), jnp.float32)
    zer64 = jnp.zeros((RPT, C), jnp.float32)

    hist = _deg_kernel(dstp, ones16, zer16).reshape(2, NP, 16)
    hs1 = _layer1(xp, hist, W1)
    t1 = _scatter_kernel(hs1, srcp, dstp, zer64).reshape(2, NP, C)
    hs2 = _mid(t1, hs1, hist, b1.reshape(1, C), W2)
    t2 = _scatter_kernel(hs2, srcp, dstp, zer64).reshape(2, NP, C)
    hs3 = _mid(t2, hs2, hist, b2.reshape(1, C), W3)
    t3 = _scatter_kernel(hs3, srcp, dstp, zer64).reshape(2, NP, C)
    return _final(t3, hs3, hist, b3.reshape(1, C), segp,
                  Wl1, bl1.reshape(1, 32), Wl2, bl2.reshape(1, 1))


# speedup vs baseline: 1.0482x; 1.0482x over previous
import functools

import jax
import jax.numpy as jnp
from jax import lax
from jax.experimental import pallas as pl
from jax.experimental.pallas import tpu as pltpu
from jax.experimental.pallas import tpu_sc as plsc

N = 10000
NP = 10240
F_IN = 128
C = 64
G = 64
NC = 2
NS = 16
NW = NC * NS
RPT = NP // NS
CHUNK = 128
NCH = 80
E_PAD = NW * NCH * CHUNK

_mesh = plsc.VectorSubcoreMesh(core_axis_name="c", subcore_axis_name="s")
_sc_params = pltpu.CompilerParams(use_tc_tiling_on_sc=False)


@functools.partial(
    pl.kernel,
    mesh=_mesh,
    out_type=jax.ShapeDtypeStruct((NC * NP, 16), jnp.float32),
    compiler_params=_sc_params,
    scratch_types=[
        pltpu.VMEM((NCH, CHUNK), jnp.int32),
        pltpu.VMEM((CHUNK, 16), jnp.float32),
        pltpu.VMEM_SHARED((NP, 16), jnp.float32),
    ],
)
def _deg_kernel(dst_hbm, ones_hbm, zeros_hbm, out_hbm, dst_v, ones_v, hist):
    c = lax.axis_index("c")
    s = lax.axis_index("s")
    wid = s * NC + c
    pltpu.sync_copy(dst_hbm.at[wid], dst_v)
    pltpu.sync_copy(ones_hbm, ones_v)
    pltpu.sync_copy(zeros_hbm, hist.at[pl.ds(s * RPT, RPT)])
    plsc.subcore_barrier()

    def body(j, carry):
        pltpu.sync_copy(ones_v, hist.at[dst_v.at[j]], add=True)
        return carry

    lax.fori_loop(0, NCH, body, 0)
    plsc.subcore_barrier()
    pltpu.sync_copy(hist.at[pl.ds(s * RPT, RPT)],
                    out_hbm.at[pl.ds(c * NP + s * RPT, RPT)])


GRP = 4
NGRP = NCH // GRP
NCHI = NCH + GRP


@functools.partial(
    pl.kernel,
    mesh=_mesh,
    out_type=jax.ShapeDtypeStruct((NC * NP, C), jnp.float32),
    compiler_params=_sc_params,
    scratch_types=[
        pltpu.VMEM((NCHI, CHUNK), jnp.int32),
        pltpu.VMEM((NCH, CHUNK), jnp.int32),
        pltpu.VMEM((CHUNK, C), jnp.float32),
        pltpu.VMEM((CHUNK, C), jnp.float32),
        pltpu.VMEM_SHARED((NP, C), jnp.float32),
        pltpu.VMEM_SHARED((NP, C), jnp.float32),
        pltpu.SemaphoreType.DMA,
        pltpu.SemaphoreType.DMA,
        pltpu.SemaphoreType.DMA,
        pltpu.SemaphoreType.DMA,
    ],
)
def _scatter_kernel(hs_hbm, src_hbm, dst_hbm, zeros_hbm, out_hbm,
                    src_v, dst_v, buf_a, buf_b, hs_s, acc,
                    sga, sgb, ssa, ssb):
    c = lax.axis_index("c")
    s = lax.axis_index("s")
    wid = s * NC + c
    pltpu.sync_copy(src_hbm.at[wid], src_v)
    pltpu.sync_copy(dst_hbm.at[wid], dst_v)
    pltpu.sync_copy(hs_hbm.at[pl.ds(s * RPT, RPT)],
                    hs_s.at[pl.ds(s * RPT, RPT)])
    pltpu.sync_copy(zeros_hbm, acc.at[pl.ds(s * RPT, RPT)])
    plsc.subcore_barrier()

    def start_g(buf, sem, j):
        pltpu.make_async_copy(hs_s.at[src_v.at[j]], buf, sem).start()

    def wait_g(buf, sem):
        pltpu.make_async_copy(hs_s.at[src_v.at[0]], buf, sem).wait()

    start_g(buf_a, sga, 0)

    def body(i, carry):
        j = 2 * i
        wait_g(buf_a, sga)
        start_g(buf_b, sgb, j + 1)
        pltpu.sync_copy(buf_a, acc.at[dst_v.at[j]], add=True)
        wait_g(buf_b, sgb)
        start_g(buf_a, sga, j + 2)
        pltpu.sync_copy(buf_b, acc.at[dst_v.at[j + 1]], add=True)
        return carry

    lax.fori_loop(0, NCH // 2, body, 0)
    wait_g(buf_a, sga)
    plsc.subcore_barrier()
    pltpu.sync_copy(acc.at[pl.ds(s * RPT, RPT)],
                    out_hbm.at[pl.ds(c * NP + s * RPT, RPT)])


def _dis(hist_ref):
    deg = hist_ref[0, :, 0:1] + hist_ref[1, :, 0:1] + 1.0
    return lax.rsqrt(deg)


def _layer1_body(x_ref, hist_ref, w_ref, o_ref):
    h = jnp.dot(x_ref[...], w_ref[...], preferred_element_type=jnp.float32)
    o_ref[...] = h * _dis(hist_ref)


def _mid_body(t_ref, hsp_ref, hist_ref, b_ref, w_ref, o_ref):
    dis = _dis(hist_ref)
    agg = dis * (t_ref[0] + t_ref[1] + hsp_ref[...]) + b_ref[...]
    h = jnp.maximum(agg, 0.0)
    o_ref[...] = jnp.dot(h, w_ref[...], preferred_element_type=jnp.float32) * dis


def _final_body(t_ref, hs3_ref, hist_ref, b3_ref, seg_ref,
                wl1_ref, bl1_ref, wl2_ref, bl2_ref, o_ref, p_acc):
    i = pl.program_id(0)

    @pl.when(i == 0)
    def _():
        p_acc[...] = jnp.zeros_like(p_acc)

    dis = _dis(hist_ref)
    h3 = dis * (t_ref[0] + t_ref[1] + hs3_ref[...]) + b3_ref[...]
    onehot = (seg_ref[...] == lax.broadcasted_iota(jnp.int32, (RPT, G), 1)
              ).astype(jnp.float32)
    p_acc[...] += lax.dot_general(onehot, h3, (((0,), (0,)), ((), ())),
                                  preferred_element_type=jnp.float32)

    @pl.when(i == pl.num_programs(0) - 1)
    def _():
        p = p_acc[...]
        pr = jnp.maximum(
            jnp.dot(p, wl1_ref[...], preferred_element_type=jnp.float32)
            + bl1_ref[...], 0.0)
        o_ref[...] = (jnp.dot(pr, wl2_ref[...],
                              preferred_element_type=jnp.float32)
                      + bl2_ref[...])


def _layer1(xp, hist, W1):
    return pl.pallas_call(
        _layer1_body,
        grid=(NS,),
        in_specs=[
            pl.BlockSpec((RPT, F_IN), lambda i: (i, 0)),
            pl.BlockSpec((2, RPT, 16), lambda i: (0, i, 0)),
            pl.BlockSpec((F_IN, C), lambda i: (0, 0)),
        ],
        out_specs=pl.BlockSpec((RPT, C), lambda i: (i, 0)),
        out_shape=jax.ShapeDtypeStruct((NP, C), jnp.float32),
    )(xp, hist, W1)


def _mid(t, hsp, hist, bias, W):
    return pl.pallas_call(
        _mid_body,
        grid=(NS,),
        in_specs=[
            pl.BlockSpec((2, RPT, C), lambda i: (0, i, 0)),
            pl.BlockSpec((RPT, C), lambda i: (i, 0)),
            pl.BlockSpec((2, RPT, 16), lambda i: (0, i, 0)),
            pl.BlockSpec((1, C), lambda i: (0, 0)),
            pl.BlockSpec((C, C), lambda i: (0, 0)),
        ],
        out_specs=pl.BlockSpec((RPT, C), lambda i: (i, 0)),
        out_shape=jax.ShapeDtypeStruct((NP, C), jnp.float32),
    )(t, hsp, hist, bias, W)


def _final(t, hs3, hist, b3, segp, Wl1, bl1, Wl2, bl2):
    return pl.pallas_call(
        _final_body,
        grid=(NS,),
        in_specs=[
            pl.BlockSpec((2, RPT, C), lambda i: (0, i, 0)),
            pl.BlockSpec((RPT, C), lambda i: (i, 0)),
            pl.BlockSpec((2, RPT, 16), lambda i: (0, i, 0)),
            pl.BlockSpec((1, C), lambda i: (0, 0)),
            pl.BlockSpec((RPT, 1), lambda i: (i, 0)),
            pl.BlockSpec((C, 32), lambda i: (0, 0)),
            pl.BlockSpec((1, 32), lambda i: (0, 0)),
            pl.BlockSpec((32, 1), lambda i: (0, 0)),
            pl.BlockSpec((1, 1), lambda i: (0, 0)),
        ],
        out_specs=pl.BlockSpec((G, 1), lambda i: (0, 0)),
        out_shape=jax.ShapeDtypeStruct((G, 1), jnp.float32),
        scratch_shapes=[pltpu.VMEM((G, C), jnp.float32)],
    )(t, hs3, hist, b3, segp, Wl1, bl1, Wl2, bl2)


def kernel(x, e, b, W1, b1, W2, b2, W3, b3, Wl1, bl1, Wl2, bl2):
    E = e.shape[1]
    xp = jnp.pad(x, ((0, NP - N), (0, 0)))
    pad = jnp.full((E_PAD - E,), N, jnp.int32)
    srcp = jnp.concatenate([e[0], pad]).reshape(NW, NCH, CHUNK)
    srcp = jnp.concatenate(
        [srcp, jnp.full((NW, NCHI - NCH, CHUNK), N, jnp.int32)], axis=1)
    dstp = jnp.concatenate([e[1], pad]).reshape(NW, NCH, CHUNK)
    segp = jnp.concatenate([b, jnp.full((NP - N,), G, jnp.int32)]
                           ).reshape(NP, 1)
    ones16 = jnp.ones((CHUNK, 16), jnp.float32)
    zer16 = jnp.zeros((RPT, 16), jnp.float32)
    zer64 = jnp.zeros((RPT, C), jnp.float32)

    hist = _deg_kernel(dstp, ones16, zer16).reshape(2, NP, 16)
    hs1 = _layer1(xp, hist, W1)
    t1 = _scatter_kernel(hs1, srcp, dstp, zer64).reshape(2, NP, C)
    hs2 = _mid(t1, hs1, hist, b1.reshape(1, C), W2)
    t2 = _scatter_kernel(hs2, srcp, dstp, zer64).reshape(2, NP, C)
    hs3 = _mid(t2, hs2, hist, b2.reshape(1, C), W3)
    t3 = _scatter_kernel(hs3, srcp, dstp, zer64).reshape(2, NP, C)
    return _final(t3, hs3, hist, b3.reshape(1, C), segp,
                  Wl1, bl1.reshape(1, 32), Wl2, bl2.reshape(1, 1))


# speedup vs baseline: 1.0783x; 1.0287x over previous
import functools

import jax
import jax.numpy as jnp
from jax import lax
from jax.experimental import pallas as pl
from jax.experimental.pallas import tpu as pltpu
from jax.experimental.pallas import tpu_sc as plsc

N = 10000
NP = 10240
F_IN = 128
C = 64
G = 64
NC = 2
NS = 16
NW = NC * NS
RPT = NP // NS
CHUNK = 128
NCH = 80
E_PAD = NW * NCH * CHUNK

_mesh = plsc.VectorSubcoreMesh(core_axis_name="c", subcore_axis_name="s")
_sc_params = pltpu.CompilerParams(use_tc_tiling_on_sc=False)


@functools.partial(
    pl.kernel,
    mesh=_mesh,
    out_type=jax.ShapeDtypeStruct((NC * NP, 16), jnp.float32),
    compiler_params=_sc_params,
    scratch_types=[
        pltpu.VMEM((NCH, CHUNK), jnp.int32),
        pltpu.VMEM((CHUNK, 16), jnp.float32),
        pltpu.VMEM_SHARED((NP, 16), jnp.float32),
    ],
)
def _deg_kernel(dst_hbm, ones_hbm, zeros_hbm, out_hbm, dst_v, ones_v, hist):
    c = lax.axis_index("c")
    s = lax.axis_index("s")
    wid = s * NC + c
    pltpu.sync_copy(dst_hbm.at[wid], dst_v)
    pltpu.sync_copy(ones_hbm, ones_v)
    pltpu.sync_copy(zeros_hbm, hist.at[pl.ds(s * RPT, RPT)])
    plsc.subcore_barrier()

    def body(j, carry):
        pltpu.sync_copy(ones_v, hist.at[dst_v.at[j]], add=True)
        return carry

    lax.fori_loop(0, NCH, body, 0)
    plsc.subcore_barrier()
    pltpu.sync_copy(hist.at[pl.ds(s * RPT, RPT)],
                    out_hbm.at[pl.ds(c * NP + s * RPT, RPT)])


GRP = 4
NGRP = NCH // GRP
NCHI = NCH + GRP


@functools.partial(
    pl.kernel,
    mesh=_mesh,
    out_type=jax.ShapeDtypeStruct((NC * NP, C), jnp.float32),
    compiler_params=_sc_params,
    scratch_types=[
        pltpu.VMEM((NCHI, CHUNK), jnp.int32),
        pltpu.VMEM((NCH, CHUNK), jnp.int32),
        pltpu.VMEM((CHUNK, C), jnp.float32),
        pltpu.VMEM((CHUNK, C), jnp.float32),
        pltpu.VMEM_SHARED((NP, C), jnp.float32),
        pltpu.VMEM_SHARED((NP, C), jnp.float32),
        pltpu.SemaphoreType.DMA,
        pltpu.SemaphoreType.DMA,
        pltpu.SemaphoreType.DMA,
        pltpu.SemaphoreType.DMA,
    ],
)
def _scatter_kernel(hs_hbm, src_hbm, dst_hbm, zeros_hbm, out_hbm,
                    src_v, dst_v, buf_a, buf_b, hs_s, acc,
                    sga, sgb, ssa, ssb):
    c = lax.axis_index("c")
    s = lax.axis_index("s")
    wid = s * NC + c
    pltpu.sync_copy(src_hbm.at[wid], src_v)
    pltpu.sync_copy(dst_hbm.at[wid], dst_v)
    pltpu.sync_copy(hs_hbm.at[pl.ds(s * RPT, RPT)],
                    hs_s.at[pl.ds(s * RPT, RPT)])
    pltpu.sync_copy(zeros_hbm, acc.at[pl.ds(s * RPT, RPT)])
    plsc.subcore_barrier()

    def start_g(buf, sem, j):
        pltpu.make_async_copy(hs_s.at[src_v.at[j]], buf, sem).start()

    def wait_g(buf, sem):
        pltpu.make_async_copy(hs_s.at[src_v.at[0]], buf, sem).wait()

    start_g(buf_a, sga, 0)

    def body(i, carry):
        j = 2 * i
        wait_g(buf_a, sga)
        start_g(buf_b, sgb, j + 1)
        pltpu.sync_copy(buf_a, acc.at[dst_v.at[j]], add=True)
        wait_g(buf_b, sgb)
        start_g(buf_a, sga, j + 2)
        pltpu.sync_copy(buf_b, acc.at[dst_v.at[j + 1]], add=True)
        return carry

    lax.fori_loop(0, NCH // 2, body, 0)
    wait_g(buf_a, sga)
    plsc.subcore_barrier()
    pltpu.sync_copy(acc.at[pl.ds(s * RPT, RPT)],
                    out_hbm.at[pl.ds(c * NP + s * RPT, RPT)])


def _layer1_body(x_ref, h0_ref, h1_ref, w_ref, hs_ref, dis_ref):
    deg = h0_ref[:, 0:1] + h1_ref[:, 0:1] + 1.0
    dis = lax.rsqrt(deg)
    dis_ref[...] = dis
    h = jnp.dot(x_ref[...], w_ref[...], preferred_element_type=jnp.float32)
    hs_ref[...] = h * dis


def _mid_body(t0_ref, t1_ref, hsp_ref, dis_ref, b_ref, w_ref, o_ref):
    dis = dis_ref[...]
    agg = dis * (t0_ref[...] + t1_ref[...] + hsp_ref[...]) + b_ref[...]
    h = jnp.maximum(agg, 0.0)
    o_ref[...] = jnp.dot(h, w_ref[...], preferred_element_type=jnp.float32) * dis


def _final_body(t0_ref, t1_ref, hs3_ref, dis_ref, b3_ref, seg_ref,
                wl1_ref, bl1_ref, wl2_ref, bl2_ref, o_ref, p_acc):
    i = pl.program_id(0)

    @pl.when(i == 0)
    def _():
        p_acc[...] = jnp.zeros_like(p_acc)

    dis = dis_ref[...]
    h3 = dis * (t0_ref[...] + t1_ref[...] + hs3_ref[...]) + b3_ref[...]
    onehot = (seg_ref[...] == lax.broadcasted_iota(jnp.int32, (RPT, G), 1)
              ).astype(jnp.float32)
    p_acc[...] += lax.dot_general(onehot, h3, (((0,), (0,)), ((), ())),
                                  preferred_element_type=jnp.float32)

    @pl.when(i == pl.num_programs(0) - 1)
    def _():
        p = p_acc[...]
        pr = jnp.maximum(
            jnp.dot(p, wl1_ref[...], preferred_element_type=jnp.float32)
            + bl1_ref[...], 0.0)
        o_ref[...] = (jnp.dot(pr, wl2_ref[...],
                              preferred_element_type=jnp.float32)
                      + bl2_ref[...])


_blk = lambda: pl.BlockSpec((RPT, C), lambda i: (i, 0))
_blk0 = lambda: pl.BlockSpec((RPT, C), lambda i: (NS + i, 0))
_dblk = lambda: pl.BlockSpec((RPT, 1), lambda i: (i, 0))


def _layer1(xp, hist, W1):
    return pl.pallas_call(
        _layer1_body,
        grid=(NS,),
        in_specs=[
            pl.BlockSpec((RPT, F_IN), lambda i: (i, 0)),
            pl.BlockSpec((RPT, 16), lambda i: (i, 0)),
            pl.BlockSpec((RPT, 16), lambda i: (NS + i, 0)),
            pl.BlockSpec((F_IN, C), lambda i: (0, 0)),
        ],
        out_specs=[_blk(), _dblk()],
        out_shape=[jax.ShapeDtypeStruct((NP, C), jnp.float32),
                   jax.ShapeDtypeStruct((NP, 1), jnp.float32)],
    )(xp, hist, hist, W1)


def _mid(t, hsp, dis, bias, W):
    return pl.pallas_call(
        _mid_body,
        grid=(NS,),
        in_specs=[
            _blk(), _blk0(), _blk(), _dblk(),
            pl.BlockSpec((1, C), lambda i: (0, 0)),
            pl.BlockSpec((C, C), lambda i: (0, 0)),
        ],
        out_specs=_blk(),
        out_shape=jax.ShapeDtypeStruct((NP, C), jnp.float32),
    )(t, t, hsp, dis, bias, W)


def _final(t, hs3, dis, b3, segp, Wl1, bl1, Wl2, bl2):
    return pl.pallas_call(
        _final_body,
        grid=(NS,),
        in_specs=[
            _blk(), _blk0(), _blk(), _dblk(),
            pl.BlockSpec((1, C), lambda i: (0, 0)),
            pl.BlockSpec((RPT, 1), lambda i: (i, 0)),
            pl.BlockSpec((C, 32), lambda i: (0, 0)),
            pl.BlockSpec((1, 32), lambda i: (0, 0)),
            pl.BlockSpec((32, 1), lambda i: (0, 0)),
            pl.BlockSpec((1, 1), lambda i: (0, 0)),
        ],
        out_specs=pl.BlockSpec((G, 1), lambda i: (0, 0)),
        out_shape=jax.ShapeDtypeStruct((G, 1), jnp.float32),
        scratch_shapes=[pltpu.VMEM((G, C), jnp.float32)],
    )(t, t, hs3, dis, b3, segp, Wl1, bl1, Wl2, bl2)


def kernel(x, e, b, W1, b1, W2, b2, W3, b3, Wl1, bl1, Wl2, bl2):
    E = e.shape[1]
    xp = jnp.pad(x, ((0, NP - N), (0, 0)))
    ep = jnp.pad(e, ((0, 0), (0, E_PAD - E)), constant_values=N)
    srcp = jnp.pad(ep[0].reshape(NW, NCH, CHUNK),
                   ((0, 0), (0, NCHI - NCH), (0, 0)), constant_values=N)
    dstp = ep[1].reshape(NW, NCH, CHUNK)
    segp = jnp.pad(b, (0, NP - N), constant_values=G).reshape(NP, 1)
    ones16 = jnp.ones((CHUNK, 16), jnp.float32)
    zer16 = jnp.zeros((RPT, 16), jnp.float32)
    zer64 = jnp.zeros((RPT, C), jnp.float32)

    hist = _deg_kernel(dstp, ones16, zer16)
    hs1, dis = _layer1(xp, hist, W1)
    t1 = _scatter_kernel(hs1, srcp, dstp, zer64)
    hs2 = _mid(t1, hs1, dis, b1.reshape(1, C), W2)
    t2 = _scatter_kernel(hs2, srcp, dstp, zer64)
    hs3 = _mid(t2, hs2, dis, b2.reshape(1, C), W3)
    t3 = _scatter_kernel(hs3, srcp, dstp, zer64)
    return _final(t3, hs3, dis, b3.reshape(1, C), segp,
                  Wl1, bl1.reshape(1, 32), Wl2, bl2.reshape(1, 1))


# speedup vs baseline: 1.1461x; 1.0628x over previous
import functools

import jax
import jax.numpy as jnp
from jax import lax
from jax.experimental import pallas as pl
from jax.experimental.pallas import tpu as pltpu
from jax.experimental.pallas import tpu_sc as plsc

N = 10000
NP = 10240
F_IN = 128
C = 64
G = 64
NC = 2
NS = 16
NW = NC * NS
RPT = NP // NS
TBR = 2560
TG = NP // TBR
CHUNK = 128
NCH = 80
E_PAD = NW * NCH * CHUNK

_mesh = plsc.VectorSubcoreMesh(core_axis_name="c", subcore_axis_name="s")
_sc_params = pltpu.CompilerParams(use_tc_tiling_on_sc=False)


@functools.partial(
    pl.kernel,
    mesh=_mesh,
    out_type=jax.ShapeDtypeStruct((NC * NP, 16), jnp.float32),
    compiler_params=_sc_params,
    scratch_types=[
        pltpu.VMEM((NCH, CHUNK), jnp.int32),
        pltpu.VMEM((CHUNK, 16), jnp.float32),
        pltpu.VMEM_SHARED((NP, 16), jnp.float32),
    ],
)
def _deg_kernel(dst_hbm, ones_hbm, zeros_hbm, out_hbm, dst_v, ones_v, hist):
    c = lax.axis_index("c")
    s = lax.axis_index("s")
    wid = s * NC + c
    pltpu.sync_copy(dst_hbm.at[wid], dst_v)
    pltpu.sync_copy(ones_hbm, ones_v)
    pltpu.sync_copy(zeros_hbm, hist.at[pl.ds(s * RPT, RPT)])
    plsc.subcore_barrier()

    def body(j, carry):
        pltpu.sync_copy(ones_v, hist.at[dst_v.at[j]], add=True)
        return carry

    lax.fori_loop(0, NCH, body, 0)
    plsc.subcore_barrier()
    pltpu.sync_copy(hist.at[pl.ds(s * RPT, RPT)],
                    out_hbm.at[pl.ds(c * NP + s * RPT, RPT)])


GRP = 4
NGRP = NCH // GRP
NCHI = NCH + GRP


@functools.partial(
    pl.kernel,
    mesh=_mesh,
    out_type=jax.ShapeDtypeStruct((NC * NP, C), jnp.float32),
    compiler_params=_sc_params,
    scratch_types=[
        pltpu.VMEM((NCHI, CHUNK), jnp.int32),
        pltpu.VMEM((NCH, CHUNK), jnp.int32),
        pltpu.VMEM((CHUNK, C), jnp.float32),
        pltpu.VMEM((CHUNK, C), jnp.float32),
        pltpu.VMEM_SHARED((NP, C), jnp.float32),
        pltpu.VMEM_SHARED((NP, C), jnp.float32),
        pltpu.SemaphoreType.DMA,
        pltpu.SemaphoreType.DMA,
        pltpu.SemaphoreType.DMA,
        pltpu.SemaphoreType.DMA,
    ],
)
def _scatter_kernel(hs_hbm, src_hbm, dst_hbm, zeros_hbm, out_hbm,
                    src_v, dst_v, buf_a, buf_b, hs_s, acc,
                    sga, sgb, ssa, ssb):
    c = lax.axis_index("c")
    s = lax.axis_index("s")
    wid = s * NC + c
    pltpu.sync_copy(src_hbm.at[wid], src_v)
    pltpu.sync_copy(dst_hbm.at[wid], dst_v)
    pltpu.sync_copy(hs_hbm.at[pl.ds(s * RPT, RPT)],
                    hs_s.at[pl.ds(s * RPT, RPT)])
    pltpu.sync_copy(zeros_hbm, acc.at[pl.ds(s * RPT, RPT)])
    plsc.subcore_barrier()

    def start_g(buf, sem, j):
        pltpu.make_async_copy(hs_s.at[src_v.at[j]], buf, sem).start()

    def wait_g(buf, sem):
        pltpu.make_async_copy(hs_s.at[src_v.at[0]], buf, sem).wait()

    start_g(buf_a, sga, 0)

    def body(i, carry):
        j = 2 * i
        wait_g(buf_a, sga)
        start_g(buf_b, sgb, j + 1)
        pltpu.sync_copy(buf_a, acc.at[dst_v.at[j]], add=True)
        wait_g(buf_b, sgb)
        start_g(buf_a, sga, j + 2)
        pltpu.sync_copy(buf_b, acc.at[dst_v.at[j + 1]], add=True)
        return carry

    lax.fori_loop(0, NCH // 2, body, 0)
    wait_g(buf_a, sga)
    plsc.subcore_barrier()
    pltpu.sync_copy(acc.at[pl.ds(s * RPT, RPT)],
                    out_hbm.at[pl.ds(c * NP + s * RPT, RPT)])


def _layer1_body(x_ref, h0_ref, h1_ref, w_ref, hs_ref, dis_ref):
    deg = h0_ref[:, 0:1] + h1_ref[:, 0:1] + 1.0
    dis = lax.rsqrt(deg)
    dis_ref[...] = dis
    h = jnp.dot(x_ref[...], w_ref[...], preferred_element_type=jnp.float32)
    hs_ref[...] = h * dis


def _mid_body(t0_ref, t1_ref, hsp_ref, dis_ref, b_ref, w_ref, o_ref):
    dis = dis_ref[...]
    agg = dis * (t0_ref[...] + t1_ref[...] + hsp_ref[...]) + b_ref[...]
    h = jnp.maximum(agg, 0.0)
    o_ref[...] = jnp.dot(h, w_ref[...], preferred_element_type=jnp.float32) * dis


def _final_body(t0_ref, t1_ref, hs3_ref, dis_ref, b3_ref, seg_ref,
                wl1_ref, bl1_ref, wl2_ref, bl2_ref, o_ref, p_acc):
    i = pl.program_id(0)

    @pl.when(i == 0)
    def _():
        p_acc[...] = jnp.zeros_like(p_acc)

    dis = dis_ref[...]
    h3 = dis * (t0_ref[...] + t1_ref[...] + hs3_ref[...]) + b3_ref[...]
    onehot = (seg_ref[...] == lax.broadcasted_iota(jnp.int32, (TBR, G), 1)
              ).astype(jnp.float32)
    p_acc[...] += lax.dot_general(onehot, h3, (((0,), (0,)), ((), ())),
                                  preferred_element_type=jnp.float32)

    @pl.when(i == pl.num_programs(0) - 1)
    def _():
        p = p_acc[...]
        pr = jnp.maximum(
            jnp.dot(p, wl1_ref[...], preferred_element_type=jnp.float32)
            + bl1_ref[...], 0.0)
        o_ref[...] = (jnp.dot(pr, wl2_ref[...],
                              preferred_element_type=jnp.float32)
                      + bl2_ref[...])


_blk = lambda: pl.BlockSpec((TBR, C), lambda i: (i, 0))
_blk0 = lambda: pl.BlockSpec((TBR, C), lambda i: (TG + i, 0))
_dblk = lambda: pl.BlockSpec((TBR, 1), lambda i: (i, 0))


def _layer1(xp, hist, W1):
    return pl.pallas_call(
        _layer1_body,
        grid=(TG,),
        in_specs=[
            pl.BlockSpec((TBR, F_IN), lambda i: (i, 0)),
            pl.BlockSpec((TBR, 16), lambda i: (i, 0)),
            pl.BlockSpec((TBR, 16), lambda i: (TG + i, 0)),
            pl.BlockSpec((F_IN, C), lambda i: (0, 0)),
        ],
        out_specs=[_blk(), _dblk()],
        out_shape=[jax.ShapeDtypeStruct((NP, C), jnp.float32),
                   jax.ShapeDtypeStruct((NP, 1), jnp.float32)],
    )(xp, hist, hist, W1)


def _mid(t, hsp, dis, bias, W):
    return pl.pallas_call(
        _mid_body,
        grid=(TG,),
        in_specs=[
            _blk(), _blk0(), _blk(), _dblk(),
            pl.BlockSpec((1, C), lambda i: (0, 0)),
            pl.BlockSpec((C, C), lambda i: (0, 0)),
        ],
        out_specs=_blk(),
        out_shape=jax.ShapeDtypeStruct((NP, C), jnp.float32),
    )(t, t, hsp, dis, bias, W)


def _final(t, hs3, dis, b3, segp, Wl1, bl1, Wl2, bl2):
    return pl.pallas_call(
        _final_body,
        grid=(TG,),
        in_specs=[
            _blk(), _blk0(), _blk(), _dblk(),
            pl.BlockSpec((1, C), lambda i: (0, 0)),
            pl.BlockSpec((TBR, 1), lambda i: (i, 0)),
            pl.BlockSpec((C, 32), lambda i: (0, 0)),
            pl.BlockSpec((1, 32), lambda i: (0, 0)),
            pl.BlockSpec((32, 1), lambda i: (0, 0)),
            pl.BlockSpec((1, 1), lambda i: (0, 0)),
        ],
        out_specs=pl.BlockSpec((G, 1), lambda i: (0, 0)),
        out_shape=jax.ShapeDtypeStruct((G, 1), jnp.float32),
        scratch_shapes=[pltpu.VMEM((G, C), jnp.float32)],
    )(t, t, hs3, dis, b3, segp, Wl1, bl1, Wl2, bl2)


def kernel(x, e, b, W1, b1, W2, b2, W3, b3, Wl1, bl1, Wl2, bl2):
    E = e.shape[1]
    xp = jnp.pad(x, ((0, NP - N), (0, 0)))
    ep = jnp.pad(e, ((0, 0), (0, E_PAD - E)), constant_values=N)
    srcp = jnp.pad(ep[0].reshape(NW, NCH, CHUNK),
                   ((0, 0), (0, NCHI - NCH), (0, 0)), constant_values=N)
    dstp = ep[1].reshape(NW, NCH, CHUNK)
    segp = jnp.pad(b, (0, NP - N), constant_values=G).reshape(NP, 1)
    ones16 = jnp.ones((CHUNK, 16), jnp.float32)
    zer16 = jnp.zeros((RPT, 16), jnp.float32)
    zer64 = jnp.zeros((RPT, C), jnp.float32)

    hist = _deg_kernel(dstp, ones16, zer16)
    hs1, dis = _layer1(xp, hist, W1)
    t1 = _scatter_kernel(hs1, srcp, dstp, zer64)
    hs2 = _mid(t1, hs1, dis, b1.reshape(1, C), W2)
    t2 = _scatter_kernel(hs2, srcp, dstp, zer64)
    hs3 = _mid(t2, hs2, dis, b2.reshape(1, C), W3)
    t3 = _scatter_kernel(hs3, srcp, dstp, zer64)
    return _final(t3, hs3, dis, b3.reshape(1, C), segp,
                  Wl1, bl1.reshape(1, 32), Wl2, bl2.reshape(1, 1))


# speedup vs baseline: 1.1570x; 1.0096x over previous
import functools

import jax
import jax.numpy as jnp
from jax import lax
from jax.experimental import pallas as pl
from jax.experimental.pallas import tpu as pltpu
from jax.experimental.pallas import tpu_sc as plsc

N = 10000
NP = 10240
F_IN = 128
C = 64
G = 64
NC = 2
NS = 16
NW = NC * NS
RPT = NP // NS
TBR = 2560
TG = NP // TBR
CHUNK = 128
NCH = 80
E_PAD = NW * NCH * CHUNK

_mesh = plsc.VectorSubcoreMesh(core_axis_name="c", subcore_axis_name="s")
_sc_params = pltpu.CompilerParams(use_tc_tiling_on_sc=False)


@functools.partial(
    pl.kernel,
    mesh=_mesh,
    out_type=jax.ShapeDtypeStruct((NC * NP, 16), jnp.float32),
    compiler_params=_sc_params,
    scratch_types=[
        pltpu.VMEM((NCH, CHUNK), jnp.int32),
        pltpu.VMEM((CHUNK, 16), jnp.float32),
        pltpu.VMEM_SHARED((NP, 16), jnp.float32),
        pltpu.SemaphoreType.DMA,
        pltpu.SemaphoreType.DMA,
    ],
)
def _deg_kernel(dst_hbm, ones_hbm, zeros_hbm, out_hbm, dst_v, ones_v, hist,
                sa, sb):
    c = lax.axis_index("c")
    s = lax.axis_index("s")
    wid = s * NC + c
    pltpu.sync_copy(dst_hbm.at[wid], dst_v)
    pltpu.sync_copy(ones_hbm, ones_v)
    pltpu.sync_copy(zeros_hbm, hist.at[pl.ds(s * RPT, RPT)])
    plsc.subcore_barrier()

    def start_s(sem, j):
        pltpu.make_async_copy(ones_v, hist.at[dst_v.at[j]],
                              sem).start(add=True)

    def wait_s(sem):
        pltpu.make_async_copy(ones_v, hist.at[dst_v.at[0]], sem).wait()

    start_s(sa, 0)

    def body(i, carry):
        j = 2 * i
        start_s(sb, j + 1)
        wait_s(sa)
        start_s(sa, j + 2)
        wait_s(sb)
        return carry

    lax.fori_loop(0, NCH // 2 - 1, body, 0)
    start_s(sb, NCH - 1)
    wait_s(sa)
    wait_s(sb)
    plsc.subcore_barrier()
    pltpu.sync_copy(hist.at[pl.ds(s * RPT, RPT)],
                    out_hbm.at[pl.ds(c * NP + s * RPT, RPT)])


GRP = 4
NGRP = NCH // GRP
NCHI = NCH + GRP


@functools.partial(
    pl.kernel,
    mesh=_mesh,
    out_type=jax.ShapeDtypeStruct((NC * NP, C), jnp.float32),
    compiler_params=_sc_params,
    scratch_types=[
        pltpu.VMEM((NCHI, CHUNK), jnp.int32),
        pltpu.VMEM((NCH, CHUNK), jnp.int32),
        pltpu.VMEM((CHUNK, C), jnp.float32),
        pltpu.VMEM((CHUNK, C), jnp.float32),
        pltpu.VMEM_SHARED((NP, C), jnp.float32),
        pltpu.VMEM_SHARED((NP, C), jnp.float32),
        pltpu.SemaphoreType.DMA,
        pltpu.SemaphoreType.DMA,
        pltpu.SemaphoreType.DMA,
        pltpu.SemaphoreType.DMA,
    ],
)
def _scatter_kernel(hs_hbm, src_hbm, dst_hbm, zeros_hbm, out_hbm,
                    src_v, dst_v, buf_a, buf_b, hs_s, acc,
                    sga, sgb, ssa, ssb):
    c = lax.axis_index("c")
    s = lax.axis_index("s")
    wid = s * NC + c
    pltpu.make_async_copy(src_hbm.at[wid], src_v, sga).start()
    pltpu.make_async_copy(dst_hbm.at[wid], dst_v, sgb).start()
    pltpu.make_async_copy(hs_hbm.at[pl.ds(s * RPT, RPT)],
                          hs_s.at[pl.ds(s * RPT, RPT)], ssa).start()
    pltpu.make_async_copy(zeros_hbm, acc.at[pl.ds(s * RPT, RPT)],
                          ssb).start()
    pltpu.make_async_copy(src_hbm.at[wid], src_v, sga).wait()
    pltpu.make_async_copy(dst_hbm.at[wid], dst_v, sgb).wait()
    pltpu.make_async_copy(hs_hbm.at[pl.ds(s * RPT, RPT)],
                          hs_s.at[pl.ds(s * RPT, RPT)], ssa).wait()
    pltpu.make_async_copy(zeros_hbm, acc.at[pl.ds(s * RPT, RPT)],
                          ssb).wait()
    plsc.subcore_barrier()

    def start_g(buf, sem, j):
        pltpu.make_async_copy(hs_s.at[src_v.at[j]], buf, sem).start()

    def wait_g(buf, sem):
        pltpu.make_async_copy(hs_s.at[src_v.at[0]], buf, sem).wait()

    start_g(buf_a, sga, 0)

    def body(i, carry):
        j = 2 * i
        wait_g(buf_a, sga)
        start_g(buf_b, sgb, j + 1)
        pltpu.sync_copy(buf_a, acc.at[dst_v.at[j]], add=True)
        wait_g(buf_b, sgb)
        start_g(buf_a, sga, j + 2)
        pltpu.sync_copy(buf_b, acc.at[dst_v.at[j + 1]], add=True)
        return carry

    lax.fori_loop(0, NCH // 2, body, 0)
    wait_g(buf_a, sga)
    plsc.subcore_barrier()
    pltpu.sync_copy(acc.at[pl.ds(s * RPT, RPT)],
                    out_hbm.at[pl.ds(c * NP + s * RPT, RPT)])


def _matmul1_body(x_ref, w_ref, u_ref):
    u_ref[...] = jnp.dot(x_ref[...], w_ref[...],
                         preferred_element_type=jnp.float32)


def _scale1_body(u_ref, h0_ref, h1_ref, hs_ref, dis_ref):
    deg = h0_ref[:, 0:1] + h1_ref[:, 0:1] + 1.0
    dis = lax.rsqrt(deg)
    dis_ref[...] = dis
    hs_ref[...] = u_ref[...] * dis


def _mid_body(t0_ref, t1_ref, hsp_ref, dis_ref, b_ref, w_ref, o_ref):
    dis = dis_ref[...]
    agg = dis * (t0_ref[...] + t1_ref[...] + hsp_ref[...]) + b_ref[...]
    h = jnp.maximum(agg, 0.0)
    o_ref[...] = jnp.dot(h, w_ref[...], preferred_element_type=jnp.float32) * dis


def _final_body(t0_ref, t1_ref, hs3_ref, dis_ref, b3_ref, seg_ref,
                wl1_ref, bl1_ref, wl2_ref, bl2_ref, o_ref, p_acc):
    i = pl.program_id(0)

    @pl.when(i == 0)
    def _():
        p_acc[...] = jnp.zeros_like(p_acc)

    dis = dis_ref[...]
    h3 = dis * (t0_ref[...] + t1_ref[...] + hs3_ref[...]) + b3_ref[...]
    onehot = (seg_ref[...] == lax.broadcasted_iota(jnp.int32, (TBR, G), 1)
              ).astype(jnp.float32)
    p_acc[...] += lax.dot_general(onehot, h3, (((0,), (0,)), ((), ())),
                                  preferred_element_type=jnp.float32)

    @pl.when(i == pl.num_programs(0) - 1)
    def _():
        p = p_acc[...]
        pr = jnp.maximum(
            jnp.dot(p, wl1_ref[...], preferred_element_type=jnp.float32)
            + bl1_ref[...], 0.0)
        o_ref[...] = (jnp.dot(pr, wl2_ref[...],
                              preferred_element_type=jnp.float32)
                      + bl2_ref[...])


_blk = lambda: pl.BlockSpec((TBR, C), lambda i: (i, 0))
_blk0 = lambda: pl.BlockSpec((TBR, C), lambda i: (TG + i, 0))
_dblk = lambda: pl.BlockSpec((TBR, 1), lambda i: (i, 0))


def _matmul1(xp, W1):
    return pl.pallas_call(
        _matmul1_body,
        grid=(TG,),
        in_specs=[
            pl.BlockSpec((TBR, F_IN), lambda i: (i, 0)),
            pl.BlockSpec((F_IN, C), lambda i: (0, 0)),
        ],
        out_specs=_blk(),
        out_shape=jax.ShapeDtypeStruct((NP, C), jnp.float32),
    )(xp, W1)


def _scale1(u1, hist):
    return pl.pallas_call(
        _scale1_body,
        grid=(TG,),
        in_specs=[
            _blk(),
            pl.BlockSpec((TBR, 16), lambda i: (i, 0)),
            pl.BlockSpec((TBR, 16), lambda i: (TG + i, 0)),
        ],
        out_specs=[_blk(), _dblk()],
        out_shape=[jax.ShapeDtypeStruct((NP, C), jnp.float32),
                   jax.ShapeDtypeStruct((NP, 1), jnp.float32)],
    )(u1, hist, hist)


def _mid(t, hsp, dis, bias, W):
    return pl.pallas_call(
        _mid_body,
        grid=(TG,),
        in_specs=[
            _blk(), _blk0(), _blk(), _dblk(),
            pl.BlockSpec((1, C), lambda i: (0, 0)),
            pl.BlockSpec((C, C), lambda i: (0, 0)),
        ],
        out_specs=_blk(),
        out_shape=jax.ShapeDtypeStruct((NP, C), jnp.float32),
    )(t, t, hsp, dis, bias, W)


def _final(t, hs3, dis, b3, segp, Wl1, bl1, Wl2, bl2):
    return pl.pallas_call(
        _final_body,
        grid=(TG,),
        in_specs=[
            _blk(), _blk0(), _blk(), _dblk(),
            pl.BlockSpec((1, C), lambda i: (0, 0)),
            pl.BlockSpec((TBR, 1), lambda i: (i, 0)),
            pl.BlockSpec((C, 32), lambda i: (0, 0)),
            pl.BlockSpec((1, 32), lambda i: (0, 0)),
            pl.BlockSpec((32, 1), lambda i: (0, 0)),
            pl.BlockSpec((1, 1), lambda i: (0, 0)),
        ],
        out_specs=pl.BlockSpec((G, 1), lambda i: (0, 0)),
        out_shape=jax.ShapeDtypeStruct((G, 1), jnp.float32),
        scratch_shapes=[pltpu.VMEM((G, C), jnp.float32)],
    )(t, t, hs3, dis, b3, segp, Wl1, bl1, Wl2, bl2)


def kernel(x, e, b, W1, b1, W2, b2, W3, b3, Wl1, bl1, Wl2, bl2):
    E = e.shape[1]
    xp = jnp.pad(x, ((0, NP - N), (0, 0)))
    ep = jnp.pad(e, ((0, 0), (0, E_PAD - E)), constant_values=N)
    srcp = jnp.pad(ep[0].reshape(NW, NCH, CHUNK),
                   ((0, 0), (0, NCHI - NCH), (0, 0)), constant_values=N)
    dstp = ep[1].reshape(NW, NCH, CHUNK)
    segp = jnp.pad(b, (0, NP - N), constant_values=G).reshape(NP, 1)
    ones16 = jnp.ones((CHUNK, 16), jnp.float32)
    zer16 = jnp.zeros((RPT, 16), jnp.float32)
    zer64 = jnp.zeros((RPT, C), jnp.float32)

    hist = _deg_kernel(dstp, ones16, zer16)
    u1 = _matmul1(xp, W1)
    hs1, dis = _scale1(u1, hist)
    t1 = _scatter_kernel(hs1, srcp, dstp, zer64)
    hs2 = _mid(t1, hs1, dis, b1.reshape(1, C), W2)
    t2 = _scatter_kernel(hs2, srcp, dstp, zer64)
    hs3 = _mid(t2, hs2, dis, b2.reshape(1, C), W3)
    t3 = _scatter_kernel(hs3, srcp, dstp, zer64)
    return _final(t3, hs3, dis, b3.reshape(1, C), segp,
                  Wl1, bl1.reshape(1, 32), Wl2, bl2.reshape(1, 1))
